# baseline (speedup 1.0000x reference)
import jax
import jax.numpy as jnp
from jax.experimental import pallas as pl
from jax.experimental.pallas import tpu as pltpu

N_EXPERTS = 8
D_MODEL = 768
D_FF = 768
N_TOKENS = 2048
THRESHOLD = 0.1
TOKEN_TILE = 2048
E_PER_STEP = 1


def _moe_body(x_ref, w1_ref, b1_ref, w2_ref, b2_ref, wg_ref, bg_ref,
              out_ref, coeff_ref):
    s = pl.program_id(0)

    @pl.when(s == 0)
    def _gate():
        xx = x_ref[:]
        logits = jnp.dot(xx, wg_ref[:], preferred_element_type=jnp.float32)
        logits = logits + bg_ref[:]
        mx = jnp.max(logits, axis=1, keepdims=True)
        ex = jnp.exp(logits - mx)
        w = ex / jnp.sum(ex, axis=1, keepdims=True)
        act = w > THRESHOLD
        act_any = jnp.any(act, axis=0, keepdims=True)
        any_act = jnp.any(act)
        coeff_ref[:] = jnp.where(any_act, w * act_any.astype(jnp.float32),
                                 1.0 / N_EXPERTS)

    def pair_tile(t):
        rows = pl.ds(t * TOKEN_TILE, TOKEN_TILE)
        x = x_ref[rows, :]
        cf_all = coeff_ref[rows, :]
        iota = jax.lax.broadcasted_iota(jnp.int32, (TOKEN_TILE, N_EXPERTS), 1)
        acc = None
        for j in range(E_PER_STEP):
            h = jnp.dot(x, w1_ref[j], preferred_element_type=jnp.float32)
            h = jnp.maximum(h + b1_ref[j], 0.0)
            o = jnp.dot(h, w2_ref[j], preferred_element_type=jnp.float32)
            o = o + b2_ref[j]
            eg = s * E_PER_STEP + j
            cf = jnp.sum(jnp.where(iota == eg, cf_all, 0.0),
                         axis=1, keepdims=True)
            term = cf * o
            acc = term if acc is None else acc + term
        return rows, acc

    n_tiles = N_TOKENS // TOKEN_TILE

    @pl.when(s == 0)
    def _first():
        for t in range(n_tiles):
            rows, term = pair_tile(t)
            out_ref[rows, :] = term

    @pl.when(s != 0)
    def _rest():
        for t in range(n_tiles):
            rows, term = pair_tile(t)
            out_ref[rows, :] = out_ref[rows, :] + term


@jax.jit
def kernel(t, x, W1, b1, W2, b2, Wg, bg):
    del t
    wg_x = Wg[:D_MODEL]
    bg2 = bg.reshape(1, N_EXPERTS)
    b1r = b1.reshape(N_EXPERTS, 1, D_FF)
    b2r = b2.reshape(N_EXPERTS, 1, D_MODEL)

    out = pl.pallas_call(
        _moe_body,
        grid=(N_EXPERTS // E_PER_STEP,),
        in_specs=[
            pl.BlockSpec((N_TOKENS, D_MODEL), lambda s: (0, 0)),
            pl.BlockSpec((E_PER_STEP, D_MODEL, D_FF), lambda s: (s, 0, 0)),
            pl.BlockSpec((E_PER_STEP, 1, D_FF), lambda s: (s, 0, 0)),
            pl.BlockSpec((E_PER_STEP, D_FF, D_MODEL), lambda s: (s, 0, 0)),
            pl.BlockSpec((E_PER_STEP, 1, D_MODEL), lambda s: (s, 0, 0)),
            pl.BlockSpec((D_MODEL, N_EXPERTS), lambda s: (0, 0)),
            pl.BlockSpec((1, N_EXPERTS), lambda s: (0, 0)),
        ],
        out_specs=pl.BlockSpec((N_TOKENS, D_MODEL), lambda s: (0, 0)),
        out_shape=jax.ShapeDtypeStruct((N_TOKENS, D_MODEL), jnp.float32),
        scratch_shapes=[
            pltpu.VMEM((N_TOKENS, N_EXPERTS), jnp.float32),
        ],
    )(x, W1, b1r, W2, b2r, wg_x, bg2)
    return out


# speedup vs baseline: 1.0174x; 1.0174x over previous
import jax
import jax.numpy as jnp
from jax.experimental import pallas as pl
from jax.experimental.pallas import tpu as pltpu

N_EXPERTS = 8
D_MODEL = 768
D_FF = 768
N_TOKENS = 2048
THRESHOLD = 0.1
TOKEN_TILE = 1024
E_PER_STEP = 1


def _moe_body(x_ref, w1_ref, b1_ref, w2_ref, b2_ref, wg_ref, bg_ref,
              out_ref, coeff_ref):
    s = pl.program_id(0)

    @pl.when(s == 0)
    def _gate():
        xx = x_ref[:]
        logits = jnp.dot(xx, wg_ref[:], preferred_element_type=jnp.float32)
        logits = logits + bg_ref[:]
        mx = jnp.max(logits, axis=1, keepdims=True)
        ex = jnp.exp(logits - mx)
        w = ex / jnp.sum(ex, axis=1, keepdims=True)
        act = w > THRESHOLD
        act_any = jnp.any(act, axis=0, keepdims=True)
        any_act = jnp.any(act)
        coeff_ref[:] = jnp.where(any_act, w * act_any.astype(jnp.float32),
                                 1.0 / N_EXPERTS)

    def pair_tile(t):
        rows = pl.ds(t * TOKEN_TILE, TOKEN_TILE)
        x = x_ref[rows, :]
        cf_all = coeff_ref[rows, :]
        iota = jax.lax.broadcasted_iota(jnp.int32, (TOKEN_TILE, N_EXPERTS), 1)
        acc = None
        for j in range(E_PER_STEP):
            h = jnp.dot(x, w1_ref[j], preferred_element_type=jnp.float32)
            h = jnp.maximum(h + b1_ref[j], 0.0)
            o = jnp.dot(h, w2_ref[j], preferred_element_type=jnp.float32)
            o = o + b2_ref[j]
            eg = s * E_PER_STEP + j
            cf = jnp.sum(jnp.where(iota == eg, cf_all, 0.0),
                         axis=1, keepdims=True)
            term = cf * o
            acc = term if acc is None else acc + term
        return rows, acc

    n_tiles = N_TOKENS // TOKEN_TILE

    @pl.when(s == 0)
    def _first():
        for t in range(n_tiles):
            rows, term = pair_tile(t)
            out_ref[rows, :] = term

    @pl.when(s != 0)
    def _rest():
        for t in range(n_tiles):
            rows, term = pair_tile(t)
            out_ref[rows, :] = out_ref[rows, :] + term


@jax.jit
def kernel(t, x, W1, b1, W2, b2, Wg, bg):
    del t
    wg_x = Wg[:D_MODEL]
    bg2 = bg.reshape(1, N_EXPERTS)
    b1r = b1.reshape(N_EXPERTS, 1, D_FF)
    b2r = b2.reshape(N_EXPERTS, 1, D_MODEL)

    out = pl.pallas_call(
        _moe_body,
        grid=(N_EXPERTS // E_PER_STEP,),
        in_specs=[
            pl.BlockSpec((N_TOKENS, D_MODEL), lambda s: (0, 0)),
            pl.BlockSpec((E_PER_STEP, D_MODEL, D_FF), lambda s: (s, 0, 0)),
            pl.BlockSpec((E_PER_STEP, 1, D_FF), lambda s: (s, 0, 0)),
            pl.BlockSpec((E_PER_STEP, D_FF, D_MODEL), lambda s: (s, 0, 0)),
            pl.BlockSpec((E_PER_STEP, 1, D_MODEL), lambda s: (s, 0, 0)),
            pl.BlockSpec((D_MODEL, N_EXPERTS), lambda s: (0, 0)),
            pl.BlockSpec((1, N_EXPERTS), lambda s: (0, 0)),
        ],
        out_specs=pl.BlockSpec((N_TOKENS, D_MODEL), lambda s: (0, 0)),
        out_shape=jax.ShapeDtypeStruct((N_TOKENS, D_MODEL), jnp.float32),
        scratch_shapes=[
            pltpu.VMEM((N_TOKENS, N_EXPERTS), jnp.float32),
        ],
    )(x, W1, b1r, W2, b2r, wg_x, bg2)
    return out


# speedup vs baseline: 1.0398x; 1.0220x over previous
import jax
import jax.numpy as jnp
from jax.experimental import pallas as pl
from jax.experimental.pallas import tpu as pltpu

N_EXPERTS = 8
D_MODEL = 768
D_FF = 768
N_TOKENS = 2048
THRESHOLD = 0.1
TOKEN_TILE = 512
E_PER_STEP = 1


def _moe_body(x_ref, w1_ref, b1_ref, w2_ref, b2_ref, wg_ref, bg_ref,
              out_ref, coeff_ref):
    s = pl.program_id(0)

    @pl.when(s == 0)
    def _gate():
        xx = x_ref[:]
        logits = jnp.dot(xx, wg_ref[:], preferred_element_type=jnp.float32)
        logits = logits + bg_ref[:]
        mx = jnp.max(logits, axis=1, keepdims=True)
        ex = jnp.exp(logits - mx)
        w = ex / jnp.sum(ex, axis=1, keepdims=True)
        act = w > THRESHOLD
        act_any = jnp.any(act, axis=0, keepdims=True)
        any_act = jnp.any(act)
        coeff_ref[:] = jnp.where(any_act, w * act_any.astype(jnp.float32),
                                 1.0 / N_EXPERTS)

    def expert_tile(t):
        rows = pl.ds(t * TOKEN_TILE, TOKEN_TILE)
        x = x_ref[rows, :]
        cf_all = coeff_ref[rows, :]
        iota = jax.lax.broadcasted_iota(jnp.int32, (TOKEN_TILE, N_EXPERTS), 1)
        h = jnp.dot(x, w1_ref[0], preferred_element_type=jnp.float32)
        h = jnp.maximum(h + b1_ref[0], 0.0)
        o = jnp.dot(h, w2_ref[0], preferred_element_type=jnp.float32)
        o = o + b2_ref[0]
        cf = jnp.sum(jnp.where(iota == s - 1, cf_all, 0.0),
                     axis=1, keepdims=True)
        return rows, cf * o

    n_tiles = N_TOKENS // TOKEN_TILE

    @pl.when(s == 1)
    def _first():
        for t in range(n_tiles):
            rows, term = expert_tile(t)
            out_ref[rows, :] = term

    @pl.when(s > 1)
    def _rest():
        for t in range(n_tiles):
            rows, term = expert_tile(t)
            out_ref[rows, :] = out_ref[rows, :] + term


@jax.jit
def kernel(t, x, W1, b1, W2, b2, Wg, bg):
    del t
    wg_x = Wg[:D_MODEL]
    bg2 = bg.reshape(1, N_EXPERTS)
    b1r = b1.reshape(N_EXPERTS, 1, D_FF)
    b2r = b2.reshape(N_EXPERTS, 1, D_MODEL)

    out = pl.pallas_call(
        _moe_body,
        grid=(N_EXPERTS + 1,),
        in_specs=[
            pl.BlockSpec((N_TOKENS, D_MODEL), lambda s: (0, 0)),
            pl.BlockSpec((1, D_MODEL, D_FF),
                         lambda s: (jnp.maximum(s - 1, 0), 0, 0)),
            pl.BlockSpec((1, 1, D_FF),
                         lambda s: (jnp.maximum(s - 1, 0), 0, 0)),
            pl.BlockSpec((1, D_FF, D_MODEL),
                         lambda s: (jnp.maximum(s - 1, 0), 0, 0)),
            pl.BlockSpec((1, 1, D_MODEL),
                         lambda s: (jnp.maximum(s - 1, 0), 0, 0)),
            pl.BlockSpec((D_MODEL, N_EXPERTS), lambda s: (0, 0)),
            pl.BlockSpec((1, N_EXPERTS), lambda s: (0, 0)),
        ],
        out_specs=pl.BlockSpec((N_TOKENS, D_MODEL), lambda s: (0, 0)),
        out_shape=jax.ShapeDtypeStruct((N_TOKENS, D_MODEL), jnp.float32),
        scratch_shapes=[
            pltpu.VMEM((N_TOKENS, N_EXPERTS), jnp.float32),
        ],
    )(x, W1, b1r, W2, b2r, wg_x, bg2)
    return out
